# Initial kernel scaffold; baseline (speedup 1.0000x reference)
import jax
import jax.numpy as jnp
from jax.experimental import pallas as pl


def kernel(x, codebook, bias, indices):
    raise NotImplementedError("write your pallas kernel here")


# speedup vs baseline: 543.1106x; 543.1106x over previous
import jax
import jax.numpy as jnp
from jax.experimental import pallas as pl
from jax.experimental.pallas import tpu as pltpu

_IN = 4096
_OUT = 4096
_CB = 256

_BN_DQ = 512
_BM = 256
_BNO = 2048


def _dequant_body(cb_ref, idx_ref, w_ref):
    cb = cb_ref[...]
    idx = idx_ref[...]
    lo = cb[:, :128]
    hi = cb[:, 128:]
    idx7 = jnp.bitwise_and(idx, 127)
    glo = jnp.take_along_axis(lo, idx7, axis=1)
    ghi = jnp.take_along_axis(hi, idx7, axis=1)
    w = jnp.where(idx < 128, glo, ghi)
    w_ref[...] = w.astype(jnp.bfloat16)


def _matmul_body(x_ref, w_ref, b_ref, o_ref):
    x = x_ref[...].astype(jnp.bfloat16)
    acc = jax.lax.dot_general(
        x, w_ref[...], (((1,), (1,)), ((), ())),
        preferred_element_type=jnp.float32)
    o_ref[...] = acc + b_ref[...]


def kernel(x, codebook, bias, indices):
    idx = indices.astype(jnp.int32)

    n_dq = _OUT // _BN_DQ
    n_dq_half = n_dq // 2
    w = pl.pallas_call(
        _dequant_body,
        grid=(2, n_dq_half),
        in_specs=[
            pl.BlockSpec((_BN_DQ, _CB), lambda c, n: (c * n_dq_half + n, 0)),
            pl.BlockSpec((_BN_DQ, _IN), lambda c, n: (c * n_dq_half + n, 0)),
        ],
        out_specs=pl.BlockSpec((_BN_DQ, _IN), lambda c, n: (c * n_dq_half + n, 0)),
        out_shape=jax.ShapeDtypeStruct((_OUT, _IN), jnp.bfloat16),
        compiler_params=pltpu.CompilerParams(
            dimension_semantics=("parallel", "arbitrary")),
        name="dequant_codebook",
    )(codebook, idx)

    b, s, _ = x.shape
    m = b * s
    xm = x.reshape(m, _IN)
    m_tiles = m // _BM // 2
    n_tiles = _OUT // _BNO

    out = pl.pallas_call(
        _matmul_body,
        grid=(2, n_tiles, m_tiles),
        in_specs=[
            pl.BlockSpec((_BM, _IN), lambda c, n, mm: (c * m_tiles + mm, 0)),
            pl.BlockSpec((_BNO, _IN), lambda c, n, mm: (n, 0)),
            pl.BlockSpec((1, _BNO), lambda c, n, mm: (0, n)),
        ],
        out_specs=pl.BlockSpec((_BM, _BNO), lambda c, n, mm: (c * m_tiles + mm, n)),
        out_shape=jax.ShapeDtypeStruct((m, _OUT), jnp.float32),
        compiler_params=pltpu.CompilerParams(
            dimension_semantics=("parallel", "arbitrary", "arbitrary")),
        name="dequant_linear_matmul",
    )(xm, w, bias.reshape(1, _OUT))
    return out.reshape(b, s, _OUT)
